# baseline (speedup 1.0000x reference)
import jax
import jax.numpy as jnp
from jax import lax
from jax.experimental import pallas as pl
from jax.experimental.pallas import tpu as pltpu
from jax.experimental.pallas import tpu_sc as plsc

_D = 128
_B = 112
_NC = 2
_NS = 16
_NW = _NC * _NS
_CW = 128
_N1 = 5000
_N2 = 2500


def _make_agg(n_tgt_pad: int, n_chunks: int):
    rows_per_sub = n_tgt_pad // _NS
    mesh = plsc.VectorSubcoreMesh(core_axis_name="c", subcore_axis_name="s")

    def body(table, src_sl, dst_sl, out_sum, out_cnt,
             src_v, dst_v, src_v1, dst_v1,
             rows_v, rows_v1, ones_v, zrow_v, acc_sh, cnt_sh,
             sem, sem1, csem, csem1):
        cid = lax.axis_index("c")
        sid = lax.axis_index("s")
        wid = cid * _NS + sid

        zero16 = jnp.zeros((16,), jnp.float32)
        one16 = jnp.ones((16,), jnp.float32)
        for i in range(16):
            for j in range(_D // 16):
                zrow_v[i, pl.ds(j * 16, 16)] = zero16
        for i in range(_B):
            for j in range(_D // 16):
                ones_v[i, pl.ds(j * 16, 16)] = one16

        zbase = sid * rows_per_sub

        @pl.loop(0, rows_per_sub // 16)
        def zloop(i):
            pltpu.sync_copy(zrow_v, acc_sh.at[pl.ds(zbase + i * 16, 16), :])
            pltpu.sync_copy(zrow_v, cnt_sh.at[pl.ds(zbase + i * 16, 16), :])

        plsc.subcore_barrier()
        cbase = wid * n_chunks

        def _stage(j, sv, dv):
            pltpu.sync_copy(src_sl.at[cbase + j], sv)
            pltpu.sync_copy(dst_sl.at[cbase + j], dv)

        half = n_chunks // 2
        _stage(0, src_v, dst_v)
        pltpu.async_copy(table.at[src_v], rows_v, sem)
        _stage(1, src_v1, dst_v1)
        pltpu.async_copy(table.at[src_v1], rows_v1, sem1)

        @pl.loop(0, half)
        def eloop(jj):
            j = jj * 2
            pltpu.make_async_copy(table.at[src_v], rows_v, sem).wait()
            pltpu.sync_copy(rows_v, acc_sh.at[dst_v], add=True)
            pltpu.async_copy(ones_v, cnt_sh.at[dst_v], csem, add=True)
            pltpu.make_async_copy(table.at[src_v1], rows_v1, sem1).wait()
            pltpu.sync_copy(rows_v1, acc_sh.at[dst_v1], add=True)
            pltpu.async_copy(ones_v, cnt_sh.at[dst_v1], csem1, add=True)

            @pl.when(jj < half - 1)
            def _():
                pltpu.make_async_copy(ones_v, cnt_sh.at[dst_v], csem).wait()
                _stage(j + 2, src_v, dst_v)
                pltpu.async_copy(table.at[src_v], rows_v, sem)
                pltpu.make_async_copy(ones_v, cnt_sh.at[dst_v1], csem1).wait()
                _stage(j + 3, src_v1, dst_v1)
                pltpu.async_copy(table.at[src_v1], rows_v1, sem1)

        pltpu.make_async_copy(ones_v, cnt_sh.at[dst_v], csem).wait()
        pltpu.make_async_copy(ones_v, cnt_sh.at[dst_v1], csem1).wait()
        plsc.subcore_barrier()

        base = wid * rows_per_sub

        @pl.loop(0, rows_per_sub // 32)
        def rloop(i):
            pltpu.sync_copy(acc_sh.at[pl.ds(zbase + i * 32, 32), :],
                            rows_v.at[pl.ds(0, 32), :])
            pltpu.sync_copy(rows_v.at[pl.ds(0, 32), :],
                            out_sum.at[pl.ds(base + i * 32, 32)])
            pltpu.sync_copy(cnt_sh.at[pl.ds(zbase + i * 32, 32), :],
                            ones_v.at[pl.ds(0, 32), :])
            pltpu.sync_copy(ones_v.at[pl.ds(0, 32), :],
                            out_cnt.at[pl.ds(base + i * 32, 32)])

    return pl.kernel(
        body,
        out_type=(
            jax.ShapeDtypeStruct((_NW * rows_per_sub, _D), jnp.float32),
            jax.ShapeDtypeStruct((_NW * rows_per_sub, _CW), jnp.float32),
        ),
        mesh=mesh,
        scratch_types=[
            pltpu.VMEM((_B,), jnp.int32),
            pltpu.VMEM((_B,), jnp.int32),
            pltpu.VMEM((_B,), jnp.int32),
            pltpu.VMEM((_B,), jnp.int32),
            pltpu.VMEM((_B, _D), jnp.float32),
            pltpu.VMEM((_B, _D), jnp.float32),
            pltpu.VMEM((_B, _D), jnp.float32),
            pltpu.VMEM((16, _D), jnp.float32),
            pltpu.VMEM_SHARED((n_tgt_pad, _D), jnp.float32),
            pltpu.VMEM_SHARED((n_tgt_pad, _D), jnp.float32),
            pltpu.SemaphoreType.DMA,
            pltpu.SemaphoreType.DMA,
            pltpu.SemaphoreType.DMA,
            pltpu.SemaphoreType.DMA,
        ],
    )


def _pad_edges(edge_index, n_chunks, dst_pad):
    e = edge_index.shape[1]
    e_pad = _NW * n_chunks * _B
    src = jnp.concatenate(
        [edge_index[0], jnp.zeros((e_pad - e,), jnp.int32)]).reshape(_NW * n_chunks, _B)
    dst = jnp.concatenate(
        [edge_index[1], jnp.full((e_pad - e,), dst_pad, jnp.int32)]).reshape(_NW * n_chunks, _B)
    return src, dst


def _dense(sum_p, cnt_p, x, w_l, w_r, b, final: bool):
    n_pad = sum_p.shape[1]
    bm = 640
    grid = (n_pad // bm,)

    def body(sum_ref, cnt_ref, x_ref, wl_ref, wr_ref, b_ref, o_ref):
        s = sum_ref[0] + sum_ref[1]
        c = cnt_ref[0, :, 0:1] + cnt_ref[1, :, 0:1]
        mean = s / jnp.clip(c, 1.0, None)
        y = (jnp.dot(mean, wl_ref[...], preferred_element_type=jnp.float32)
             + jnp.dot(x_ref[...], wr_ref[...], preferred_element_type=jnp.float32)
             + b_ref[...])
        if final:
            m = jnp.max(y, axis=-1, keepdims=True)
            z = y - m
            o_ref[...] = z - jnp.log(jnp.sum(jnp.exp(z), axis=-1, keepdims=True))
        else:
            o_ref[...] = jnp.maximum(y, 0.0)

    return pl.pallas_call(
        body,
        grid=grid,
        in_specs=[
            pl.BlockSpec((_NC, bm, _D), lambda i: (0, i, 0)),
            pl.BlockSpec((_NC, bm, _CW), lambda i: (0, i, 0)),
            pl.BlockSpec((bm, _D), lambda i: (i, 0)),
            pl.BlockSpec((_D, _D), lambda i: (0, 0)),
            pl.BlockSpec((_D, _D), lambda i: (0, 0)),
            pl.BlockSpec((1, _D), lambda i: (0, 0)),
        ],
        out_specs=pl.BlockSpec((bm, _D), lambda i: (i, 0)),
        out_shape=jax.ShapeDtypeStruct((n_pad, _D), jnp.float32),
    )(sum_p, cnt_p, x, w_l, w_r, b)


def kernel(x, edge_index1, edge_index2, W_l1, b_l1, W_r1, W_l2, b_l2, W_r2):
    n1p = 5120
    n2p = 2560
    ch1 = -(-edge_index1.shape[1] // (_NW * _B))
    ch1 += ch1 % 2
    ch2 = -(-edge_index2.shape[1] // (_NW * _B))
    ch2 += ch2 % 2

    src1, dst1 = _pad_edges(edge_index1, ch1, n1p - 1)
    sum1, cnt1 = _make_agg(n1p, ch1)(x, src1, dst1)
    h = _dense(sum1.reshape(_NC, n1p, _D), cnt1.reshape(_NC, n1p, _CW),
               x, W_l1, W_r1, b_l1.reshape(1, _D), final=False)

    src2, dst2 = _pad_edges(edge_index2, ch2, n2p - 1)
    sum2, cnt2 = _make_agg(n2p, ch2)(h, src2, dst2)
    out = _dense(sum2.reshape(_NC, n2p, _D), cnt2.reshape(_NC, n2p, _CW),
                 h, W_l2, W_r2, b_l2.reshape(1, _D), final=True)
    return out[:_N2]


# speedup vs baseline: 1.0788x; 1.0788x over previous
import jax
import jax.numpy as jnp
from jax import lax
from jax.experimental import pallas as pl
from jax.experimental.pallas import tpu as pltpu
from jax.experimental.pallas import tpu_sc as plsc

_D = 128
_B = 112
_NC = 2
_NS = 16
_NW = _NC * _NS
_CW = 128
_N1 = 5000
_N2 = 2500


def _make_agg(n_tgt_pad: int, n_chunks: int):
    rows_per_sub = n_tgt_pad // _NS
    mesh = plsc.VectorSubcoreMesh(core_axis_name="c", subcore_axis_name="s")

    def body(table, src_sl, dst_sl, out_sum, out_cnt,
             src_v, dst_v, src_v1, dst_v1,
             rows_v, rows_v1, ones_v, zrow_v, acc_sh, cnt_sh, sem, sem1):
        cid = lax.axis_index("c")
        sid = lax.axis_index("s")
        wid = cid * _NS + sid

        zero16 = jnp.zeros((16,), jnp.float32)
        one16 = jnp.ones((16,), jnp.float32)
        for i in range(16):
            for j in range(_D // 16):
                zrow_v[i, pl.ds(j * 16, 16)] = zero16
        for i in range(_B):
            for j in range(_D // 16):
                ones_v[i, pl.ds(j * 16, 16)] = one16

        zbase = sid * rows_per_sub

        @pl.loop(0, rows_per_sub // 16)
        def zloop(i):
            pltpu.sync_copy(zrow_v, acc_sh.at[pl.ds(zbase + i * 16, 16), :])
            pltpu.sync_copy(zrow_v, cnt_sh.at[pl.ds(zbase + i * 16, 16), :])

        plsc.subcore_barrier()
        cbase = wid * n_chunks

        def _stage(j, sv, dv):
            pltpu.sync_copy(src_sl.at[cbase + j], sv)
            pltpu.sync_copy(dst_sl.at[cbase + j], dv)

        half = n_chunks // 2
        _stage(0, src_v, dst_v)
        pltpu.async_copy(table.at[src_v], rows_v, sem)

        @pl.loop(0, half)
        def eloop(jj):
            j = jj * 2
            _stage(j + 1, src_v1, dst_v1)
            pltpu.async_copy(table.at[src_v1], rows_v1, sem1)
            pltpu.make_async_copy(table.at[src_v], rows_v, sem).wait()
            pltpu.sync_copy(rows_v, acc_sh.at[dst_v], add=True)
            pltpu.sync_copy(ones_v, cnt_sh.at[dst_v], add=True)

            @pl.when(jj < half - 1)
            def _():
                _stage(j + 2, src_v, dst_v)
                pltpu.async_copy(table.at[src_v], rows_v, sem)

            pltpu.make_async_copy(table.at[src_v1], rows_v1, sem1).wait()
            pltpu.sync_copy(rows_v1, acc_sh.at[dst_v1], add=True)
            pltpu.sync_copy(ones_v, cnt_sh.at[dst_v1], add=True)

        plsc.subcore_barrier()

        base = wid * rows_per_sub

        @pl.loop(0, rows_per_sub // 32)
        def rloop(i):
            pltpu.sync_copy(acc_sh.at[pl.ds(zbase + i * 32, 32), :],
                            rows_v.at[pl.ds(0, 32), :])
            pltpu.sync_copy(rows_v.at[pl.ds(0, 32), :],
                            out_sum.at[pl.ds(base + i * 32, 32)])
            pltpu.sync_copy(cnt_sh.at[pl.ds(zbase + i * 32, 32), :],
                            ones_v.at[pl.ds(0, 32), :])
            pltpu.sync_copy(ones_v.at[pl.ds(0, 32), :],
                            out_cnt.at[pl.ds(base + i * 32, 32)])

    return pl.kernel(
        body,
        out_type=(
            jax.ShapeDtypeStruct((_NW * rows_per_sub, _D), jnp.float32),
            jax.ShapeDtypeStruct((_NW * rows_per_sub, _CW), jnp.float32),
        ),
        mesh=mesh,
        scratch_types=[
            pltpu.VMEM((_B,), jnp.int32),
            pltpu.VMEM((_B,), jnp.int32),
            pltpu.VMEM((_B,), jnp.int32),
            pltpu.VMEM((_B,), jnp.int32),
            pltpu.VMEM((_B, _D), jnp.float32),
            pltpu.VMEM((_B, _D), jnp.float32),
            pltpu.VMEM((_B, _D), jnp.float32),
            pltpu.VMEM((16, _D), jnp.float32),
            pltpu.VMEM_SHARED((n_tgt_pad, _D), jnp.float32),
            pltpu.VMEM_SHARED((n_tgt_pad, _D), jnp.float32),
            pltpu.SemaphoreType.DMA,
            pltpu.SemaphoreType.DMA,
        ],
    )


def _pad_edges(edge_index, n_chunks, dst_pad):
    e = edge_index.shape[1]
    e_pad = _NW * n_chunks * _B
    src = jnp.concatenate(
        [edge_index[0], jnp.zeros((e_pad - e,), jnp.int32)]).reshape(_NW * n_chunks, _B)
    dst = jnp.concatenate(
        [edge_index[1], jnp.full((e_pad - e,), dst_pad, jnp.int32)]).reshape(_NW * n_chunks, _B)
    return src, dst


def _dense(sum_p, cnt_p, x, w_l, w_r, b, final: bool):
    n_pad = sum_p.shape[1]
    bm = 640
    grid = (n_pad // bm,)

    def body(sum_ref, cnt_ref, x_ref, wl_ref, wr_ref, b_ref, o_ref):
        s = sum_ref[0] + sum_ref[1]
        c = cnt_ref[0, :, 0:1] + cnt_ref[1, :, 0:1]
        mean = s / jnp.clip(c, 1.0, None)
        y = (jnp.dot(mean, wl_ref[...], preferred_element_type=jnp.float32)
             + jnp.dot(x_ref[...], wr_ref[...], preferred_element_type=jnp.float32)
             + b_ref[...])
        if final:
            m = jnp.max(y, axis=-1, keepdims=True)
            z = y - m
            o_ref[...] = z - jnp.log(jnp.sum(jnp.exp(z), axis=-1, keepdims=True))
        else:
            o_ref[...] = jnp.maximum(y, 0.0)

    return pl.pallas_call(
        body,
        grid=grid,
        in_specs=[
            pl.BlockSpec((_NC, bm, _D), lambda i: (0, i, 0)),
            pl.BlockSpec((_NC, bm, _CW), lambda i: (0, i, 0)),
            pl.BlockSpec((bm, _D), lambda i: (i, 0)),
            pl.BlockSpec((_D, _D), lambda i: (0, 0)),
            pl.BlockSpec((_D, _D), lambda i: (0, 0)),
            pl.BlockSpec((1, _D), lambda i: (0, 0)),
        ],
        out_specs=pl.BlockSpec((bm, _D), lambda i: (i, 0)),
        out_shape=jax.ShapeDtypeStruct((n_pad, _D), jnp.float32),
    )(sum_p, cnt_p, x, w_l, w_r, b)


def kernel(x, edge_index1, edge_index2, W_l1, b_l1, W_r1, W_l2, b_l2, W_r2):
    n1p = 5120
    n2p = 2560
    ch1 = -(-edge_index1.shape[1] // (_NW * _B))
    ch1 += ch1 % 2
    ch2 = -(-edge_index2.shape[1] // (_NW * _B))
    ch2 += ch2 % 2

    src1, dst1 = _pad_edges(edge_index1, ch1, n1p - 1)
    sum1, cnt1 = _make_agg(n1p, ch1)(x, src1, dst1)
    h = _dense(sum1.reshape(_NC, n1p, _D), cnt1.reshape(_NC, n1p, _CW),
               x, W_l1, W_r1, b_l1.reshape(1, _D), final=False)

    src2, dst2 = _pad_edges(edge_index2, ch2, n2p - 1)
    sum2, cnt2 = _make_agg(n2p, ch2)(h, src2, dst2)
    out = _dense(sum2.reshape(_NC, n2p, _D), cnt2.reshape(_NC, n2p, _CW),
                 h, W_l2, W_r2, b_l2.reshape(1, _D), final=True)
    return out[:_N2]
